# baseline (speedup 1.0000x reference)
import functools

import jax
import jax.numpy as jnp
from jax import lax
from jax.experimental import pallas as pl
from jax.experimental.pallas import tpu as pltpu
from jax.experimental.pallas import tpu_sc as plsc

N = 10000
E = 320000
D = 128

NC = 2
NS = 16
NW = NC * NS
KM = 128
KQ = 32
NSUPER = 5
NMETA = 16 * NSUPER
NM0 = 48
NM1 = 2 * NMETA - NM0
EPAD = NS * (NM0 + NM1) * KM
NROW = N + 8
PK = 1 << 14
WOC = 40
NOC = N // WOC
OC_PER_TILE = (NOC + NS - 1) // NS
OLEN = 48


def _sc_body(x_hbm, packed_hbm, attr_hbm, out_hbm, cnt_hbm,
             idx_v, packed_v, attr_v,
             col_v0, col_v1, col_v2, col_v3,
             row_v0, row_v1, row_v2, row_v3,
             gbuf0, gbuf1, gbuf2, gbuf3, zbuf, ones_v,
             acc, acc_cnt, gsem0, gsem1, gsem2, gsem3, esem):
    cid = lax.axis_index("c")
    sid = lax.axis_index("s")
    w = cid * NS + sid
    cols = (col_v0, col_v1, col_v2, col_v3)
    rows = (row_v0, row_v1, row_v2, row_v3)
    gbufs = (gbuf0, gbuf1, gbuf2, gbuf3)
    gsems = (gsem0, gsem1, gsem2, gsem3)

    zeros = jnp.zeros((16,), jnp.float32)

    def zrow(j, _):
        for v in range(D // 16):
            zbuf[j, pl.ds(v * 16, 16)] = zeros
        return 0

    lax.fori_loop(0, WOC, zrow, 0)
    for v in range(OLEN // 16):
        ones_v[pl.ds(v * 16, 16)] = zeros

    def zchunk(i, _):
        oc = sid + i * NS

        @pl.when(oc < NOC)
        def _():
            pltpu.sync_copy(zbuf, acc.at[pl.ds(oc * WOC, WOC), :])
            pltpu.sync_copy(ones_v.at[pl.ds(0, WOC)],
                            acc_cnt.at[pl.ds(oc * WOC, WOC)])

        return 0

    lax.fori_loop(0, OC_PER_TILE, zchunk, 0)

    @pl.when(sid == 0)
    def _():
        pltpu.sync_copy(ones_v.at[pl.ds(0, 8)], acc_cnt.at[pl.ds(N, 8)])

    ones = jnp.ones((16,), jnp.float32)
    for v in range(OLEN // 16):
        ones_v[pl.ds(v * 16, 16)] = ones

    plsc.subcore_barrier()

    base_meta = jnp.where(cid == 0, sid * NM0, NS * NM0 + sid * NM1)
    nsuper = jnp.where(cid == 0, NM0 // 16, NM1 // 16)

    def superchunk(s, _):
        idx_v[...] = base_meta + s * 16 + lax.iota(jnp.int32, 16)
        ep = pltpu.async_copy(packed_hbm.at[idx_v], packed_v, esem)
        ea = pltpu.async_copy(attr_hbm.at[idx_v], attr_v, esem)
        ep.wait()
        ea.wait()

        def upk(r, _):
            for g in range(KM // 16):
                p = packed_v[r, pl.ds(g * 16, 16)]
                h = pl.ds((g % (KQ // 16)) * 16, 16)
                rows[g // (KQ // 16)][r, h] = lax.shift_right_logical(
                    p, jnp.int32(14))
                cols[g // (KQ // 16)][r, h] = lax.bitwise_and(
                    p, jnp.int32(PK - 1))
            return 0

        lax.fori_loop(0, 16, upk, 0)

        for q in range(3):
            pltpu.async_copy(x_hbm.at[cols[q].at[0]], gbufs[q], gsems[q])

        def metarow(r, _):
            for q in range(4):
                gb = gbufs[q]
                pltpu.make_async_copy(x_hbm.at[cols[q].at[r]], gb,
                                      gsems[q]).wait()
                qn = (q + 3) % 4
                rn = r + (0 if q == 0 else 1)

                @pl.when(rn < 16)
                def _():
                    pltpu.async_copy(x_hbm.at[cols[qn].at[rn]], gbufs[qn],
                                     gsems[qn])

                def group(g, _):
                    av = attr_v[r, pl.ds(q * KQ + g * 16, 16)]
                    for l in range(16):
                        sp = jnp.full((16,), av[l], dtype=jnp.float32)
                        j = g * 16 + l
                        for v in range(D // 16):
                            vsl = pl.ds(v * 16, 16)
                            gb[j, vsl] = gb[j, vsl] * sp
                    return 0

                lax.fori_loop(0, KQ // 16, group, 0)
                pltpu.sync_copy(gb, acc.at[rows[q].at[r]], add=True)
                pltpu.sync_copy(ones_v.at[pl.ds(0, KQ)],
                                acc_cnt.at[rows[q].at[r]], add=True)
            return 0

        lax.fori_loop(0, 16, metarow, 0)
        return 0

    lax.fori_loop(0, nsuper, superchunk, 0)

    plsc.subcore_barrier()

    def wchunk(i, _):
        oc = sid + i * NS

        @pl.when(oc < NOC)
        def _():
            pltpu.sync_copy(acc.at[pl.ds(oc * WOC, WOC), :],
                            out_hbm.at[cid, pl.ds(oc * WOC, WOC), :])
            pltpu.sync_copy(acc_cnt.at[pl.ds(oc * WOC, WOC)],
                            ones_v.at[pl.ds(0, WOC)])
            pltpu.sync_copy(ones_v.at[pl.ds(0, WOC)],
                            cnt_hbm.at[pl.ds(cid * N + oc * WOC, WOC)])

        return 0

    lax.fori_loop(0, OC_PER_TILE, wchunk, 0)


@functools.cache
def _sc_call():
  return pl.kernel(
    _sc_body,
    out_type=(jax.ShapeDtypeStruct((NC, N, D), jnp.float32),
              jax.ShapeDtypeStruct((NC * N,), jnp.float32)),
    mesh=plsc.VectorSubcoreMesh(core_axis_name="c", subcore_axis_name="s",
                                num_cores=NC, num_subcores=NS),
    scratch_types=[
        pltpu.VMEM((16,), jnp.int32),
        pltpu.VMEM((16, KM), jnp.int32),
        pltpu.VMEM((16, KM), jnp.float32),
        pltpu.VMEM((16, KQ), jnp.int32),
        pltpu.VMEM((16, KQ), jnp.int32),
        pltpu.VMEM((16, KQ), jnp.int32),
        pltpu.VMEM((16, KQ), jnp.int32),
        pltpu.VMEM((16, KQ), jnp.int32),
        pltpu.VMEM((16, KQ), jnp.int32),
        pltpu.VMEM((16, KQ), jnp.int32),
        pltpu.VMEM((16, KQ), jnp.int32),
        pltpu.VMEM((KQ, D), jnp.float32),
        pltpu.VMEM((KQ, D), jnp.float32),
        pltpu.VMEM((KQ, D), jnp.float32),
        pltpu.VMEM((KQ, D), jnp.float32),
        pltpu.VMEM((WOC, D), jnp.float32),
        pltpu.VMEM((OLEN,), jnp.float32),
        pltpu.VMEM_SHARED((NROW, D), jnp.float32),
        pltpu.VMEM_SHARED((NROW,), jnp.float32),
        pltpu.SemaphoreType.DMA,
        pltpu.SemaphoreType.DMA,
        pltpu.SemaphoreType.DMA,
        pltpu.SemaphoreType.DMA,
        pltpu.SemaphoreType.DMA,
    ],
  )


TB = 1000


def _tc_body(acc_ref, cnt_ref, x_ref, emb_ref, wm0_ref, wm1_ref, bm_ref,
             wu0_ref, wu1_ref, bu_ref, o_ref):
    a = acc_ref[...]
    s = a[0] + a[1]
    c = cnt_ref[...]
    cnt = c[0] + c[1]
    x_agg = s / jnp.maximum(cnt, 1.0)
    m = jnp.maximum(
        jnp.dot(x_agg, wm0_ref[...], preferred_element_type=jnp.float32)
        + jnp.dot(emb_ref[...], wm1_ref[...], preferred_element_type=jnp.float32)
        + bm_ref[...], 0.0)
    o = jnp.maximum(
        jnp.dot(x_ref[...], wu0_ref[...], preferred_element_type=jnp.float32)
        + jnp.dot(m, wu1_ref[...], preferred_element_type=jnp.float32)
        + bu_ref[...], 0.0)
    o_ref[...] = o


_tc_call = pl.pallas_call(
    _tc_body,
    grid=(N // TB,),
    in_specs=[
        pl.BlockSpec((NC, TB, D), lambda i: (0, i, 0)),
        pl.BlockSpec((NC, TB, 1), lambda i: (0, i, 0)),
        pl.BlockSpec((TB, D), lambda i: (i, 0)),
        pl.BlockSpec((TB, D), lambda i: (i, 0)),
        pl.BlockSpec((D, D), lambda i: (0, 0)),
        pl.BlockSpec((D, D), lambda i: (0, 0)),
        pl.BlockSpec((1, D), lambda i: (0, 0)),
        pl.BlockSpec((D, D), lambda i: (0, 0)),
        pl.BlockSpec((D, D), lambda i: (0, 0)),
        pl.BlockSpec((1, D), lambda i: (0, 0)),
    ],
    out_specs=pl.BlockSpec((TB, D), lambda i: (i, 0)),
    out_shape=jax.ShapeDtypeStruct((N, D), jnp.float32),
)


def kernel(x, edge_index, edge_attr, x_agg_emb, W_msg, b_msg, W_upd, b_upd):
    packed = edge_index[1] * PK + edge_index[0]
    pad = jnp.full((EPAD - E,), N * PK, dtype=jnp.int32)
    packed = jnp.concatenate([packed, pad]).reshape(NW * NMETA, KM)
    attr = jnp.concatenate(
        [edge_attr, jnp.zeros((EPAD - E,), dtype=jnp.float32)]
    ).reshape(NW * NMETA, KM)
    acc, cnt = _sc_call()(x, packed, attr)
    cnt = cnt.reshape(NC, N, 1)
    return _tc_call(acc, cnt, x, x_agg_emb,
                    W_msg[:D], W_msg[D:], b_msg.reshape(1, D),
                    W_upd[:D], W_upd[D:], b_upd.reshape(1, D))


# speedup vs baseline: 1.1964x; 1.1964x over previous
import functools

import jax
import jax.numpy as jnp
from jax import lax
from jax.experimental import pallas as pl
from jax.experimental.pallas import tpu as pltpu
from jax.experimental.pallas import tpu_sc as plsc

N = 10000
E = 320000
D = 128

NC = 2
NS = 16
NW = NC * NS
KM = 128
KQ = 32
NSUPER = 5
NMETA = 16 * NSUPER
NM0 = 112
NM1 = 2 * NMETA - NM0
EPAD = NS * (NM0 + NM1) * KM
NROW = N + 8
PK = 1 << 14
WOC = 40
NOC = N // WOC
OC_PER_TILE = (NOC + NS - 1) // NS
OLEN = 48


def _sc_body(x_hbm, packed_hbm, attr_hbm, out_hbm, cnt_hbm,
             idx_v, packed_v, attr_v,
             col_v0, col_v1, col_v2, col_v3,
             row_v0, row_v1, row_v2, row_v3,
             gbuf0, gbuf1, gbuf2, gbuf3, zbuf, ones_v,
             acc, acc_cnt, gsem0, gsem1, gsem2, gsem3, esem):
    cid = lax.axis_index("c")
    sid = lax.axis_index("s")
    w = cid * NS + sid
    cols = (col_v0, col_v1, col_v2, col_v3)
    rows = (row_v0, row_v1, row_v2, row_v3)
    gbufs = (gbuf0, gbuf1, gbuf2, gbuf3)
    gsems = (gsem0, gsem1, gsem2, gsem3)

    zeros = jnp.zeros((16,), jnp.float32)

    def zrow(j, _):
        for v in range(D // 16):
            zbuf[j, pl.ds(v * 16, 16)] = zeros
        return 0

    lax.fori_loop(0, WOC, zrow, 0)
    for v in range(OLEN // 16):
        ones_v[pl.ds(v * 16, 16)] = zeros

    def zchunk(i, _):
        oc = sid + i * NS

        @pl.when(oc < NOC)
        def _():
            pltpu.sync_copy(zbuf, acc.at[pl.ds(oc * WOC, WOC), :])
            pltpu.sync_copy(ones_v.at[pl.ds(0, WOC)],
                            acc_cnt.at[pl.ds(oc * WOC, WOC)])

        return 0

    lax.fori_loop(0, OC_PER_TILE, zchunk, 0)

    @pl.when(sid == 0)
    def _():
        pltpu.sync_copy(ones_v.at[pl.ds(0, 8)], acc_cnt.at[pl.ds(N, 8)])

    ones = jnp.ones((16,), jnp.float32)
    for v in range(OLEN // 16):
        ones_v[pl.ds(v * 16, 16)] = ones

    plsc.subcore_barrier()

    base_meta = jnp.where(cid == 0, sid * NM0, NS * NM0 + sid * NM1)
    nsuper = jnp.where(cid == 0, NM0 // 16, NM1 // 16)

    def superchunk(s, _):
        idx_v[...] = base_meta + s * 16 + lax.iota(jnp.int32, 16)
        ep = pltpu.async_copy(packed_hbm.at[idx_v], packed_v, esem)
        ea = pltpu.async_copy(attr_hbm.at[idx_v], attr_v, esem)
        ep.wait()
        ea.wait()

        def upk(r, _):
            for g in range(KM // 16):
                p = packed_v[r, pl.ds(g * 16, 16)]
                h = pl.ds((g % (KQ // 16)) * 16, 16)
                rows[g // (KQ // 16)][r, h] = lax.shift_right_logical(
                    p, jnp.int32(14))
                cols[g // (KQ // 16)][r, h] = lax.bitwise_and(
                    p, jnp.int32(PK - 1))
            return 0

        lax.fori_loop(0, 16, upk, 0)

        for q in range(3):
            pltpu.async_copy(x_hbm.at[cols[q].at[0]], gbufs[q], gsems[q])

        def metarow(r, _):
            for q in range(4):
                gb = gbufs[q]
                pltpu.make_async_copy(x_hbm.at[cols[q].at[r]], gb,
                                      gsems[q]).wait()
                qn = (q + 3) % 4
                rn = r + (0 if q == 0 else 1)

                @pl.when(rn < 16)
                def _():
                    pltpu.async_copy(x_hbm.at[cols[qn].at[rn]], gbufs[qn],
                                     gsems[qn])

                def group(g, _):
                    av = attr_v[r, pl.ds(q * KQ + g * 16, 16)]
                    for l in range(16):
                        sp = jnp.full((16,), av[l], dtype=jnp.float32)
                        j = g * 16 + l
                        for v in range(D // 16):
                            vsl = pl.ds(v * 16, 16)
                            gb[j, vsl] = gb[j, vsl] * sp
                    return 0

                lax.fori_loop(0, KQ // 16, group, 0)
                pltpu.sync_copy(gb, acc.at[rows[q].at[r]], add=True)
                pltpu.sync_copy(ones_v.at[pl.ds(0, KQ)],
                                acc_cnt.at[rows[q].at[r]], add=True)
            return 0

        lax.fori_loop(0, 16, metarow, 0)
        return 0

    lax.fori_loop(0, nsuper, superchunk, 0)

    plsc.subcore_barrier()

    def wchunk(i, _):
        oc = sid + i * NS

        @pl.when(oc < NOC)
        def _():
            pltpu.sync_copy(acc.at[pl.ds(oc * WOC, WOC), :],
                            out_hbm.at[cid, pl.ds(oc * WOC, WOC), :])
            pltpu.sync_copy(acc_cnt.at[pl.ds(oc * WOC, WOC)],
                            ones_v.at[pl.ds(0, WOC)])
            pltpu.sync_copy(ones_v.at[pl.ds(0, WOC)],
                            cnt_hbm.at[pl.ds(cid * N + oc * WOC, WOC)])

        return 0

    lax.fori_loop(0, OC_PER_TILE, wchunk, 0)


@functools.cache
def _sc_call():
  return pl.kernel(
    _sc_body,
    out_type=(jax.ShapeDtypeStruct((NC, N, D), jnp.float32),
              jax.ShapeDtypeStruct((NC * N,), jnp.float32)),
    mesh=plsc.VectorSubcoreMesh(core_axis_name="c", subcore_axis_name="s",
                                num_cores=NC, num_subcores=NS),
    scratch_types=[
        pltpu.VMEM((16,), jnp.int32),
        pltpu.VMEM((16, KM), jnp.int32),
        pltpu.VMEM((16, KM), jnp.float32),
        pltpu.VMEM((16, KQ), jnp.int32),
        pltpu.VMEM((16, KQ), jnp.int32),
        pltpu.VMEM((16, KQ), jnp.int32),
        pltpu.VMEM((16, KQ), jnp.int32),
        pltpu.VMEM((16, KQ), jnp.int32),
        pltpu.VMEM((16, KQ), jnp.int32),
        pltpu.VMEM((16, KQ), jnp.int32),
        pltpu.VMEM((16, KQ), jnp.int32),
        pltpu.VMEM((KQ, D), jnp.float32),
        pltpu.VMEM((KQ, D), jnp.float32),
        pltpu.VMEM((KQ, D), jnp.float32),
        pltpu.VMEM((KQ, D), jnp.float32),
        pltpu.VMEM((WOC, D), jnp.float32),
        pltpu.VMEM((OLEN,), jnp.float32),
        pltpu.VMEM_SHARED((NROW, D), jnp.float32),
        pltpu.VMEM_SHARED((NROW,), jnp.float32),
        pltpu.SemaphoreType.DMA,
        pltpu.SemaphoreType.DMA,
        pltpu.SemaphoreType.DMA,
        pltpu.SemaphoreType.DMA,
        pltpu.SemaphoreType.DMA,
    ],
  )


TB = 1000


def _tc_body(acc_ref, cnt_ref, x_ref, emb_ref, wm0_ref, wm1_ref, bm_ref,
             wu0_ref, wu1_ref, bu_ref, o_ref):
    a = acc_ref[...]
    s = a[0] + a[1]
    c = cnt_ref[...]
    cnt = c[0] + c[1]
    x_agg = s / jnp.maximum(cnt, 1.0)
    m = jnp.maximum(
        jnp.dot(x_agg, wm0_ref[...], preferred_element_type=jnp.float32)
        + jnp.dot(emb_ref[...], wm1_ref[...], preferred_element_type=jnp.float32)
        + bm_ref[...], 0.0)
    o = jnp.maximum(
        jnp.dot(x_ref[...], wu0_ref[...], preferred_element_type=jnp.float32)
        + jnp.dot(m, wu1_ref[...], preferred_element_type=jnp.float32)
        + bu_ref[...], 0.0)
    o_ref[...] = o


_tc_call = pl.pallas_call(
    _tc_body,
    grid=(N // TB,),
    in_specs=[
        pl.BlockSpec((NC, TB, D), lambda i: (0, i, 0)),
        pl.BlockSpec((NC, TB, 1), lambda i: (0, i, 0)),
        pl.BlockSpec((TB, D), lambda i: (i, 0)),
        pl.BlockSpec((TB, D), lambda i: (i, 0)),
        pl.BlockSpec((D, D), lambda i: (0, 0)),
        pl.BlockSpec((D, D), lambda i: (0, 0)),
        pl.BlockSpec((1, D), lambda i: (0, 0)),
        pl.BlockSpec((D, D), lambda i: (0, 0)),
        pl.BlockSpec((D, D), lambda i: (0, 0)),
        pl.BlockSpec((1, D), lambda i: (0, 0)),
    ],
    out_specs=pl.BlockSpec((TB, D), lambda i: (i, 0)),
    out_shape=jax.ShapeDtypeStruct((N, D), jnp.float32),
)


def kernel(x, edge_index, edge_attr, x_agg_emb, W_msg, b_msg, W_upd, b_upd):
    packed = edge_index[1] * PK + edge_index[0]
    pad = jnp.full((EPAD - E,), N * PK, dtype=jnp.int32)
    packed = jnp.concatenate([packed, pad]).reshape(NW * NMETA, KM)
    attr = jnp.concatenate(
        [edge_attr, jnp.zeros((EPAD - E,), dtype=jnp.float32)]
    ).reshape(NW * NMETA, KM)
    acc, cnt = _sc_call()(x, packed, attr)
    cnt = cnt.reshape(NC, N, 1)
    return _tc_call(acc, cnt, x, x_agg_emb,
                    W_msg[:D], W_msg[D:], b_msg.reshape(1, D),
                    W_upd[:D], W_upd[D:], b_upd.reshape(1, D))


# speedup vs baseline: 1.2139x; 1.0146x over previous
import functools

import jax
import jax.numpy as jnp
from jax import lax
from jax.experimental import pallas as pl
from jax.experimental.pallas import tpu as pltpu
from jax.experimental.pallas import tpu_sc as plsc

N = 10000
E = 320000
D = 128

NC = 2
NS = 16
NW = NC * NS
KM = 128
KQ = 32
NSUPER = 5
NMETA = 16 * NSUPER
NM0 = 128
NM1 = 2 * NMETA - NM0
EPAD = NS * (NM0 + NM1) * KM
NROW = N + 8
PK = 1 << 14
WOC = 40
NOC = N // WOC
OC_PER_TILE = (NOC + NS - 1) // NS
OLEN = 48


def _sc_body(x_hbm, packed_hbm, attr_hbm, out_hbm, cnt_hbm,
             idx_v, packed_v, attr_v,
             col_v0, col_v1, col_v2, col_v3,
             row_v0, row_v1, row_v2, row_v3,
             gbuf0, gbuf1, gbuf2, gbuf3, zbuf, ones_v,
             acc, acc_cnt, gsem0, gsem1, gsem2, gsem3, esem):
    cid = lax.axis_index("c")
    sid = lax.axis_index("s")
    w = cid * NS + sid
    cols = (col_v0, col_v1, col_v2, col_v3)
    rows = (row_v0, row_v1, row_v2, row_v3)
    gbufs = (gbuf0, gbuf1, gbuf2, gbuf3)
    gsems = (gsem0, gsem1, gsem2, gsem3)

    zeros = jnp.zeros((16,), jnp.float32)

    def zrow(j, _):
        for v in range(D // 16):
            zbuf[j, pl.ds(v * 16, 16)] = zeros
        return 0

    lax.fori_loop(0, WOC, zrow, 0)
    for v in range(OLEN // 16):
        ones_v[pl.ds(v * 16, 16)] = zeros

    def zchunk(i, _):
        oc = sid + i * NS

        @pl.when(oc < NOC)
        def _():
            pltpu.sync_copy(zbuf, acc.at[pl.ds(oc * WOC, WOC), :])
            pltpu.sync_copy(ones_v.at[pl.ds(0, WOC)],
                            acc_cnt.at[pl.ds(oc * WOC, WOC)])

        return 0

    lax.fori_loop(0, OC_PER_TILE, zchunk, 0)

    @pl.when(sid == 0)
    def _():
        pltpu.sync_copy(ones_v.at[pl.ds(0, 8)], acc_cnt.at[pl.ds(N, 8)])

    ones = jnp.ones((16,), jnp.float32)
    for v in range(OLEN // 16):
        ones_v[pl.ds(v * 16, 16)] = ones

    plsc.subcore_barrier()

    base_meta = jnp.where(cid == 0, sid * NM0, NS * NM0 + sid * NM1)
    nsuper = jnp.where(cid == 0, NM0 // 16, NM1 // 16)

    def superchunk(s, _):
        idx_v[...] = base_meta + s * 16 + lax.iota(jnp.int32, 16)
        ep = pltpu.async_copy(packed_hbm.at[idx_v], packed_v, esem)
        ea = pltpu.async_copy(attr_hbm.at[idx_v], attr_v, esem)
        ep.wait()
        ea.wait()

        def upk(r, _):
            for g in range(KM // 16):
                p = packed_v[r, pl.ds(g * 16, 16)]
                h = pl.ds((g % (KQ // 16)) * 16, 16)
                rows[g // (KQ // 16)][r, h] = lax.shift_right_logical(
                    p, jnp.int32(14))
                cols[g // (KQ // 16)][r, h] = lax.bitwise_and(
                    p, jnp.int32(PK - 1))
            return 0

        lax.fori_loop(0, 16, upk, 0)

        for q in range(3):
            pltpu.async_copy(x_hbm.at[cols[q].at[0]], gbufs[q], gsems[q])

        def metarow(r, _):
            for q in range(4):
                gb = gbufs[q]
                pltpu.make_async_copy(x_hbm.at[cols[q].at[r]], gb,
                                      gsems[q]).wait()
                qn = (q + 3) % 4
                rn = r + (0 if q == 0 else 1)

                @pl.when(rn < 16)
                def _():
                    pltpu.async_copy(x_hbm.at[cols[qn].at[rn]], gbufs[qn],
                                     gsems[qn])

                def group(g, _):
                    av = attr_v[r, pl.ds(q * KQ + g * 16, 16)]
                    for l in range(16):
                        sp = jnp.full((16,), av[l], dtype=jnp.float32)
                        j = g * 16 + l
                        for v in range(D // 16):
                            vsl = pl.ds(v * 16, 16)
                            gb[j, vsl] = gb[j, vsl] * sp
                    return 0

                lax.fori_loop(0, KQ // 16, group, 0)
                pltpu.sync_copy(gb, acc.at[rows[q].at[r]], add=True)
                pltpu.sync_copy(ones_v.at[pl.ds(0, KQ)],
                                acc_cnt.at[rows[q].at[r]], add=True)
            return 0

        lax.fori_loop(0, 16, metarow, 0)
        return 0

    lax.fori_loop(0, nsuper, superchunk, 0)

    plsc.subcore_barrier()

    def wchunk(i, _):
        oc = sid + i * NS

        @pl.when(oc < NOC)
        def _():
            pltpu.sync_copy(acc.at[pl.ds(oc * WOC, WOC), :],
                            out_hbm.at[cid, pl.ds(oc * WOC, WOC), :])
            pltpu.sync_copy(acc_cnt.at[pl.ds(oc * WOC, WOC)],
                            ones_v.at[pl.ds(0, WOC)])
            pltpu.sync_copy(ones_v.at[pl.ds(0, WOC)],
                            cnt_hbm.at[pl.ds(cid * N + oc * WOC, WOC)])

        return 0

    lax.fori_loop(0, OC_PER_TILE, wchunk, 0)


@functools.cache
def _sc_call():
  return pl.kernel(
    _sc_body,
    out_type=(jax.ShapeDtypeStruct((NC, N, D), jnp.float32),
              jax.ShapeDtypeStruct((NC * N,), jnp.float32)),
    mesh=plsc.VectorSubcoreMesh(core_axis_name="c", subcore_axis_name="s",
                                num_cores=NC, num_subcores=NS),
    scratch_types=[
        pltpu.VMEM((16,), jnp.int32),
        pltpu.VMEM((16, KM), jnp.int32),
        pltpu.VMEM((16, KM), jnp.float32),
        pltpu.VMEM((16, KQ), jnp.int32),
        pltpu.VMEM((16, KQ), jnp.int32),
        pltpu.VMEM((16, KQ), jnp.int32),
        pltpu.VMEM((16, KQ), jnp.int32),
        pltpu.VMEM((16, KQ), jnp.int32),
        pltpu.VMEM((16, KQ), jnp.int32),
        pltpu.VMEM((16, KQ), jnp.int32),
        pltpu.VMEM((16, KQ), jnp.int32),
        pltpu.VMEM((KQ, D), jnp.float32),
        pltpu.VMEM((KQ, D), jnp.float32),
        pltpu.VMEM((KQ, D), jnp.float32),
        pltpu.VMEM((KQ, D), jnp.float32),
        pltpu.VMEM((WOC, D), jnp.float32),
        pltpu.VMEM((OLEN,), jnp.float32),
        pltpu.VMEM_SHARED((NROW, D), jnp.float32),
        pltpu.VMEM_SHARED((NROW,), jnp.float32),
        pltpu.SemaphoreType.DMA,
        pltpu.SemaphoreType.DMA,
        pltpu.SemaphoreType.DMA,
        pltpu.SemaphoreType.DMA,
        pltpu.SemaphoreType.DMA,
    ],
  )


TB = 1000


def _tc_body(acc_ref, cnt_ref, x_ref, emb_ref, wm0_ref, wm1_ref, bm_ref,
             wu0_ref, wu1_ref, bu_ref, o_ref):
    a = acc_ref[...]
    s = a[0] + a[1]
    c = cnt_ref[...]
    cnt = c[0] + c[1]
    x_agg = s / jnp.maximum(cnt, 1.0)
    m = jnp.maximum(
        jnp.dot(x_agg, wm0_ref[...], preferred_element_type=jnp.float32)
        + jnp.dot(emb_ref[...], wm1_ref[...], preferred_element_type=jnp.float32)
        + bm_ref[...], 0.0)
    o = jnp.maximum(
        jnp.dot(x_ref[...], wu0_ref[...], preferred_element_type=jnp.float32)
        + jnp.dot(m, wu1_ref[...], preferred_element_type=jnp.float32)
        + bu_ref[...], 0.0)
    o_ref[...] = o


_tc_call = pl.pallas_call(
    _tc_body,
    grid=(N // TB,),
    in_specs=[
        pl.BlockSpec((NC, TB, D), lambda i: (0, i, 0)),
        pl.BlockSpec((NC, TB, 1), lambda i: (0, i, 0)),
        pl.BlockSpec((TB, D), lambda i: (i, 0)),
        pl.BlockSpec((TB, D), lambda i: (i, 0)),
        pl.BlockSpec((D, D), lambda i: (0, 0)),
        pl.BlockSpec((D, D), lambda i: (0, 0)),
        pl.BlockSpec((1, D), lambda i: (0, 0)),
        pl.BlockSpec((D, D), lambda i: (0, 0)),
        pl.BlockSpec((D, D), lambda i: (0, 0)),
        pl.BlockSpec((1, D), lambda i: (0, 0)),
    ],
    out_specs=pl.BlockSpec((TB, D), lambda i: (i, 0)),
    out_shape=jax.ShapeDtypeStruct((N, D), jnp.float32),
)


def kernel(x, edge_index, edge_attr, x_agg_emb, W_msg, b_msg, W_upd, b_upd):
    packed = edge_index[1] * PK + edge_index[0]
    pad = jnp.full((EPAD - E,), N * PK, dtype=jnp.int32)
    packed = jnp.concatenate([packed, pad]).reshape(NW * NMETA, KM)
    attr = jnp.concatenate(
        [edge_attr, jnp.zeros((EPAD - E,), dtype=jnp.float32)]
    ).reshape(NW * NMETA, KM)
    acc, cnt = _sc_call()(x, packed, attr)
    cnt = cnt.reshape(NC, N, 1)
    return _tc_call(acc, cnt, x, x_agg_emb,
                    W_msg[:D], W_msg[D:], b_msg.reshape(1, D),
                    W_upd[:D], W_upd[D:], b_upd.reshape(1, D))
